# baseline (speedup 1.0000x reference)
import math

import jax
import jax.numpy as jnp
from jax import lax
from jax.experimental import pallas as pl
from jax.experimental.pallas import tpu as pltpu
from jax.experimental.pallas import tpu_sc as plsc

D = 32
SCALE = math.sqrt(float(D))
NW = 32
BS = 128


def _body(xt_hbm, tbl_hbm, outA_hbm, idxs, rows0, rows1, outb, sem0, sem1):
    T = xt_hbm.shape[0]
    wid = lax.axis_index("s") * 2 + lax.axis_index("c")
    b0 = pl.multiple_of(wid * BS, BS)

    pltpu.sync_copy(xt_hbm.at[:, pl.ds(b0, BS)], idxs)

    iota16 = lax.iota(jnp.int32, 16)
    db_h = [(iota16 + 16 * h) >> 3 for h in range(2)]
    ds_h = [(iota16 + 16 * h) & 7 for h in range(2)]
    bufs = ((rows0, sem0), (rows1, sem1))

    pltpu.async_copy(tbl_hbm.at[idxs.at[0]], rows0, sem0)

    @pl.loop(0, T // 2)
    def _pair(tt):
        for p in range(2):
            t = tt * 2 + p
            rows, sem = bufs[p]
            rows_n, sem_n = bufs[1 - p]

            @pl.when(t + 1 < T)
            def _():
                pltpu.async_copy(tbl_hbm.at[idxs.at[t + 1]], rows_n, sem_n)

            pltpu.make_async_copy(tbl_hbm.at[idxs.at[t]], rows, sem).wait()

            @plsc.parallel_loop(0, BS, unroll=4)
            def _r(r):
                rv = jnp.full((16,), 0, jnp.int32) + r
                for h in range(2):
                    vals = rows[r, h * 16:(h + 1) * 16] * SCALE
                    plsc.store_scatter(outb, [db_h[h], ds_h[h], rv], vals)

            pltpu.sync_copy(outb, outA_hbm.at[t, :, wid, :, :])


VC = 512
NCH = 1954
TAIL_V0 = 999936


def _tr_body(tt_hbm, tail_hbm, tr_hbm, vb0, vb1, ob, sem0, sem1):
    wid = lax.axis_index("s") * 2 + lax.axis_index("c")
    iota16 = lax.iota(jnp.int32, 16)
    jbase = iota16 >> 2
    colbase = (iota16 & 3) * 32
    bufs = ((vb0, sem0), (vb1, sem1))

    @pl.when(wid == 0)
    def _():
        pltpu.sync_copy(tail_hbm, tr_hbm.at[pl.ds((NCH - 1) * 128, 16)])

    def issue(c, vb, sem):
        pltpu.async_copy(tt_hbm.at[:, pl.ds(c * VC, VC)], vb, sem)

    def work(c, vb, sem):
        pltpu.make_async_copy(
            tt_hbm.at[:, pl.ds(c * VC, VC)], vb, sem).wait()

        @plsc.parallel_loop(0, D, unroll=4)
        def _d(d):
            colv = colbase + d
            for c16 in range(VC // 16):
                vals = vb[d, c16 * 16:(c16 + 1) * 16]
                plsc.store_scatter(ob, [jbase + c16 * 4, colv], vals)

        pltpu.sync_copy(ob, tr_hbm.at[pl.ds(c * 128, 128)])

    issue(wid, vb0, sem0)

    @pl.loop(0, (NCH - 1 + NW - 1) // NW // 2 + 1)
    def _pair(ii):
        for p in range(2):
            i = ii * 2 + p
            c = wid + NW * i
            vb, sem = bufs[p]
            vb_n, sem_n = bufs[1 - p]
            c_n = wid + NW * (i + 1)

            @pl.when(c_n < NCH - 1)
            def _():
                issue(c_n, vb_n, sem_n)

            @pl.when(c < NCH - 1)
            def _():
                work(c, vb, sem)


def _regroup_table(table):
    v, d = table.shape
    tt = jnp.transpose(table)
    tail = jnp.reshape(table[TAIL_V0:], (16, 128))
    mesh = plsc.VectorSubcoreMesh(core_axis_name="c", subcore_axis_name="s")
    return pl.kernel(
        _tr_body,
        out_type=jax.ShapeDtypeStruct((v * d // 128, 128), jnp.float32),
        mesh=mesh,
        scratch_types=[
            pltpu.VMEM((d, VC), jnp.float32),
            pltpu.VMEM((d, VC), jnp.float32),
            pltpu.VMEM((128, 128), jnp.float32),
            pltpu.SemaphoreType.DMA,
            pltpu.SemaphoreType.DMA,
        ],
        compiler_params=pltpu.CompilerParams(needs_layout_passes=False, disable_bounds_checks=True),
    )(tt, tail)


def kernel(x, table):
    b, t = x.shape
    v, d = table.shape
    xt = jnp.transpose(x)
    tr = _regroup_table(table)
    tbl2 = jnp.reshape(tr, (v, d))
    mesh = plsc.VectorSubcoreMesh(core_axis_name="c", subcore_axis_name="s")
    outA = pl.kernel(
        _body,
        out_type=jax.ShapeDtypeStruct((t, 4, NW, 8, 128), jnp.float32),
        mesh=mesh,
        scratch_types=[
            pltpu.VMEM((t, BS), jnp.int32),
            pltpu.VMEM((BS, d), jnp.float32),
            pltpu.VMEM((BS, d), jnp.float32),
            pltpu.VMEM((4, 8, 128), jnp.float32),
            pltpu.SemaphoreType.DMA,
            pltpu.SemaphoreType.DMA,
        ],
        compiler_params=pltpu.CompilerParams(
            needs_layout_passes=False, use_tc_tiling_on_sc=False,
            disable_bounds_checks=True),
    )(xt, tbl2)
    return jnp.reshape(jnp.transpose(outA, (2, 4, 0, 1, 3)), (b, t, d))


# speedup vs baseline: 1.0819x; 1.0819x over previous
import math

import jax
import jax.numpy as jnp
from jax import lax
from jax.experimental import pallas as pl
from jax.experimental.pallas import tpu as pltpu
from jax.experimental.pallas import tpu_sc as plsc

D = 32
SCALE = math.sqrt(float(D))
NW = 32
BS = 128


def _body(xt_hbm, tbl_hbm, outA_hbm, idxs, rows0, rows1, outb, sem0, sem1):
    T = xt_hbm.shape[0]
    wid = lax.axis_index("s") * 2 + lax.axis_index("c")
    b0 = pl.multiple_of(wid * BS, BS)

    pltpu.sync_copy(xt_hbm.at[:, pl.ds(b0, BS)], idxs)

    iota16 = lax.iota(jnp.int32, 16)
    row_gs = [iota16 + 16 * g for g in range(BS // 16)]
    bufs = ((rows0, sem0), (rows1, sem1))

    pltpu.async_copy(tbl_hbm.at[idxs.at[0]], rows0, sem0)

    @pl.loop(0, T // 2)
    def _pair(tt):
        for p in range(2):
            t = tt * 2 + p
            rows, sem = bufs[p]
            rows_n, sem_n = bufs[1 - p]

            @pl.when(t + 1 < T)
            def _():
                pltpu.async_copy(tbl_hbm.at[idxs.at[t + 1]], rows_n, sem_n)

            pltpu.make_async_copy(tbl_hbm.at[idxs.at[t]], rows, sem).wait()

            @plsc.parallel_loop(0, D, unroll=4)
            def _d(d):
                dv = jnp.full((16,), 0, jnp.int32) + d
                for g in range(BS // 16):
                    vals = plsc.load_gather(rows, [row_gs[g], dv])
                    outb[d >> 3, d & 7, g * 16:(g + 1) * 16] = vals * SCALE

            pltpu.sync_copy(outb, outA_hbm.at[t, :, wid, :, :])


VC = 512
NCH = 1954
TAIL_V0 = 999936


def _tr_body(tt_hbm, tail_hbm, tr_hbm, vb0, vb1, ob, sem0, sem1):
    wid = lax.axis_index("s") * 2 + lax.axis_index("c")
    iota16 = lax.iota(jnp.int32, 16)
    dvecs = [iota16, iota16 + 16]
    bufs = ((vb0, sem0), (vb1, sem1))

    @pl.when(wid == 0)
    def _():
        pltpu.sync_copy(tail_hbm, tr_hbm.at[pl.ds((NCH - 1) * 128, 16)])

    def issue(c, vb, sem):
        pltpu.async_copy(tt_hbm.at[:, pl.ds(c * VC, VC)], vb, sem)

    def work(c, vb, sem):
        pltpu.make_async_copy(
            tt_hbm.at[:, pl.ds(c * VC, VC)], vb, sem).wait()

        @plsc.parallel_loop(0, 128, unroll=2)
        def _jj(jj):
            s0 = jj * 4
            for k in range(4):
                vvec = jnp.full((16,), 0, jnp.int32) + (s0 + k)
                for h in range(2):
                    vals = plsc.load_gather(vb, [dvecs[h], vvec])
                    ob[jj, k * 32 + h * 16:k * 32 + h * 16 + 16] = vals

        pltpu.sync_copy(ob, tr_hbm.at[pl.ds(c * 128, 128)])

    issue(wid, vb0, sem0)

    @pl.loop(0, (NCH - 1 + NW - 1) // NW // 2 + 1)
    def _pair(ii):
        for p in range(2):
            i = ii * 2 + p
            c = wid + NW * i
            vb, sem = bufs[p]
            vb_n, sem_n = bufs[1 - p]
            c_n = wid + NW * (i + 1)

            @pl.when(c_n < NCH - 1)
            def _():
                issue(c_n, vb_n, sem_n)

            @pl.when(c < NCH - 1)
            def _():
                work(c, vb, sem)


def _regroup_table(table):
    v, d = table.shape
    tt = jnp.transpose(table)
    tail = jnp.reshape(table[TAIL_V0:], (16, 128))
    mesh = plsc.VectorSubcoreMesh(core_axis_name="c", subcore_axis_name="s")
    return pl.kernel(
        _tr_body,
        out_type=jax.ShapeDtypeStruct((v * d // 128, 128), jnp.float32),
        mesh=mesh,
        scratch_types=[
            pltpu.VMEM((d, VC), jnp.float32),
            pltpu.VMEM((d, VC), jnp.float32),
            pltpu.VMEM((128, 128), jnp.float32),
            pltpu.SemaphoreType.DMA,
            pltpu.SemaphoreType.DMA,
        ],
        compiler_params=pltpu.CompilerParams(needs_layout_passes=False),
    )(tt, tail)


def kernel(x, table):
    b, t = x.shape
    v, d = table.shape
    xt = jnp.transpose(x)
    tr = _regroup_table(table)
    tbl2 = jnp.reshape(tr, (v, d))
    mesh = plsc.VectorSubcoreMesh(core_axis_name="c", subcore_axis_name="s")
    outA = pl.kernel(
        _body,
        out_type=jax.ShapeDtypeStruct((t, 4, NW, 8, 128), jnp.float32),
        mesh=mesh,
        scratch_types=[
            pltpu.VMEM((t, BS), jnp.int32),
            pltpu.VMEM((BS, d), jnp.float32),
            pltpu.VMEM((BS, d), jnp.float32),
            pltpu.VMEM((4, 8, 128), jnp.float32),
            pltpu.SemaphoreType.DMA,
            pltpu.SemaphoreType.DMA,
        ],
        compiler_params=pltpu.CompilerParams(
            needs_layout_passes=False, use_tc_tiling_on_sc=False),
    )(xt, tbl2)
    return jnp.reshape(jnp.transpose(outA, (2, 4, 0, 1, 3)), (b, t, d))
